# baseline (speedup 1.0000x reference)
import functools

import jax
import jax.numpy as jnp
from jax.experimental import pallas as pl
from jax.experimental.pallas import tpu as pltpu
from jax.experimental.pallas import tpu_sc as plsc

_NNZ = 5
_ROWS = 3
_COLS = 4

_MESH = plsc.ScalarSubcoreMesh(axis_name="c", num_cores=1)


@functools.partial(
    pl.kernel,
    out_type=jax.ShapeDtypeStruct((_ROWS,), jnp.float32),
    mesh=_MESH,
    compiler_params=pltpu.CompilerParams(needs_layout_passes=False),
    scratch_types=[
        pltpu.SMEM((_COLS,), jnp.float32),
        pltpu.SMEM((_NNZ,), jnp.float32),
        pltpu.SMEM((2, _NNZ), jnp.int32),
        pltpu.SMEM((_ROWS,), jnp.float32),
        pltpu.SemaphoreType.DMA,
        pltpu.SemaphoreType.DMA,
        pltpu.SemaphoreType.DMA,
    ],
)
def _spmv_scs(x_hbm, v_hbm, ij_hbm, out_hbm, x_s, v_s, ij_s, o_s, s0, s1, s2):
    cp_x = pltpu.async_copy(x_hbm, x_s, s0)
    cp_v = pltpu.async_copy(v_hbm, v_s, s1)
    cp_ij = pltpu.async_copy(ij_hbm, ij_s, s2)
    for r in range(_ROWS):
        o_s[r] = jnp.float32(0.0)
    cp_x.wait()
    cp_v.wait()
    cp_ij.wait()
    for k in range(_NNZ):
        i = ij_s[0, k]
        j = ij_s[1, k]
        o_s[i] = o_s[i] + v_s[k] * x_s[j]
    pltpu.sync_copy(o_s, out_hbm)


def kernel(x, values, indices):
    out = _spmv_scs(x, values, indices.astype(jnp.int32))
    return out[:, None]


# speedup vs baseline: 1.0098x; 1.0098x over previous
import functools

import jax
import jax.numpy as jnp
from jax.experimental import pallas as pl
from jax.experimental.pallas import tpu as pltpu
from jax.experimental.pallas import tpu_sc as plsc

_NNZ = 5
_ROWS = 3
_COLS = 4

_MESH = plsc.ScalarSubcoreMesh(axis_name="c", num_cores=1)


@functools.partial(
    pl.kernel,
    out_type=jax.ShapeDtypeStruct((_ROWS,), jnp.float32),
    mesh=_MESH,
    compiler_params=pltpu.CompilerParams(
        needs_layout_passes=False,
        skip_device_barrier=True,
        disable_bounds_checks=True,
        disable_semaphore_checks=True,
    ),
    scratch_types=[
        pltpu.SMEM((_COLS,), jnp.float32),
        pltpu.SMEM((_NNZ,), jnp.float32),
        pltpu.SMEM((2, _NNZ), jnp.int32),
        pltpu.SMEM((_ROWS,), jnp.float32),
        pltpu.SemaphoreType.DMA,
        pltpu.SemaphoreType.DMA,
        pltpu.SemaphoreType.DMA,
    ],
)
def _spmv_scs(x_hbm, v_hbm, ij_hbm, out_hbm, x_s, v_s, ij_s, o_s, s0, s1, s2):
    cp_x = pltpu.async_copy(x_hbm, x_s, s0)
    cp_v = pltpu.async_copy(v_hbm, v_s, s1)
    cp_ij = pltpu.async_copy(ij_hbm, ij_s, s2)
    for r in range(_ROWS):
        o_s[r] = jnp.float32(0.0)
    cp_x.wait()
    cp_v.wait()
    cp_ij.wait()
    for k in range(_NNZ):
        i = ij_s[0, k]
        j = ij_s[1, k]
        o_s[i] = o_s[i] + v_s[k] * x_s[j]
    pltpu.sync_copy(o_s, out_hbm)


def kernel(x, values, indices):
    out = _spmv_scs(x, values, indices.astype(jnp.int32))
    return out[:, None]
